# baseline (speedup 1.0000x reference)
import functools

import jax
import jax.numpy as jnp
from jax import lax
from jax.experimental import pallas as pl
from jax.experimental.pallas import tpu as pltpu
from jax.experimental.pallas import tpu_sc as plsc

_EMB = 32
_NC = 2
_NS = 16
_NW = _NC * _NS
_SUB = 128
_BLK = 1024
_SCH = 256
_GPR = 128 // _EMB


@functools.lru_cache(maxsize=None)
def _build(n):
    bpw = n // _NW
    nblk = bpw // _BLK
    npair = (nblk - 1) // 2

    mesh = plsc.VectorSubcoreMesh(core_axis_name="c", subcore_axis_name="s",
                                  num_cores=_NC, num_subcores=_NS)

    @functools.partial(
        pl.kernel,
        mesh=mesh,
        out_type=jax.ShapeDtypeStruct((n // _GPR, 128), jnp.float32),
        scratch_types=[
            pltpu.VMEM((8, _SUB), jnp.int32),
            pltpu.VMEM((8, _SUB), jnp.int32),
            pltpu.VMEM((8, _SUB), jnp.int32),
            pltpu.VMEM((_BLK,), jnp.int32),
            pltpu.VMEM((_SCH, 128), jnp.float32),
            pltpu.VMEM((_SCH, 128), jnp.float32),
            pltpu.VMEM((_SCH // _GPR, 128), jnp.float32),
            pltpu.VMEM((_SCH // _GPR, 128), jnp.float32),
            pltpu.SemaphoreType.DMA,
            pltpu.SemaphoreType.DMA,
            pltpu.SemaphoreType.DMA,
            pltpu.SemaphoreType.DMA,
            pltpu.SemaphoreType.DMA,
        ],
        compiler_params=pltpu.CompilerParams(needs_layout_passes=False),
    )
    def gather_kernel(idx_hbm, table_hbm, out_hbm,
                      idxA, idxB, gidx, mcol, grp0, grp1, outv0, outv1,
                      isem, gsem0, gsem1, osem0, osem1):
        wid = lax.axis_index("s") * _NC + lax.axis_index("c")
        brow = wid * (bpw // _SUB)
        orow0 = wid * (bpw // _GPR)
        idx_v = (idxA, idxB)
        grp_v = (grp0, grp1)
        outv = (outv0, outv1)
        gsem = (gsem0, gsem1)
        osem = (osem0, osem1)
        iota = lax.iota(jnp.int32, 16)

        def prefetch_idx(b, islot):
            row = pl.multiple_of(brow + jnp.minimum(b, nblk - 1) * 8, 8)
            pltpu.async_copy(idx_hbm.at[pl.ds(row, 8)], idx_v[islot], isem)

        def wait_idx(islot):
            pltpu.make_async_copy(idx_hbm.at[pl.ds(0, 8)], idx_v[islot],
                                  isem).wait()

        def transform(islot):
            src = idx_v[islot]
            for r in range(8):
                for c in range(0, _SUB, 16):
                    v = src[r, pl.ds(c, 16)]
                    gidx[r, pl.ds(c, 16)] = lax.shift_right_logical(v, 2)
                    mcol[pl.ds(r * _SUB + c, 16)] = lax.shift_left(
                        lax.bitwise_and(v, 3), 5)

        def fire_gathers(j):
            s = j % 2
            for h in range(2):
                pltpu.async_copy(table_hbm.at[gidx.at[2 * j + h]],
                                 grp_v[s].at[pl.ds(h * _SUB, _SUB)], gsem[s])

        def wait_gathers(s):
            pltpu.make_async_copy(table_hbm.at[pl.ds(0, _SCH)], grp_v[s],
                                  gsem[s]).wait()

        def drain_out(s):
            pltpu.make_async_copy(outv[s], out_hbm.at[pl.ds(0, _SCH // _GPR)],
                                  osem[s]).wait()

        def extract(j):
            s = j % 2
            grp = grp_v[s]
            out_s = outv[s]
            mbase = j * _SCH

            def row_body(it, carry):
                for k in range(4):
                    r = it * 4 + k
                    r_vec = jnp.full((16,), 0, jnp.int32) + r
                    m = plsc.load_gather(mcol, [r_vec + mbase])
                    lo = plsc.load_gather(grp, [r_vec, m + iota])
                    hi = plsc.load_gather(grp, [r_vec, m + (iota + 16)])
                    out_s[it, pl.ds(k * 32, 16)] = lo
                    out_s[it, pl.ds(k * 32 + 16, 16)] = hi
                return carry

            lax.fori_loop(0, _SCH // 4, row_body, 0)

        def fire_out(b, j):
            s = j % 2
            off = pl.multiple_of(orow0 + b * (_BLK // _GPR)
                                 + j * (_SCH // _GPR), _SCH // _GPR)
            pltpu.async_copy(outv[s], out_hbm.at[pl.ds(off, _SCH // _GPR)],
                             osem[s])

        def run_block(b, islot, first):
            transform(islot)
            fire_gathers(0)
            for j in range(_BLK // _SCH):
                if j + 1 < _BLK // _SCH:
                    fire_gathers(j + 1)
                wait_gathers(j % 2)
                if not (first and j < 2):
                    drain_out(j % 2)
                extract(j)
                fire_out(b, j)

        prefetch_idx(0, 0)
        wait_idx(0)
        prefetch_idx(1, 1)
        run_block(0, 0, True)

        def pair_body(t, carry):
            for bb in range(2):
                b = 1 + 2 * t + bb
                islot = (1 + bb) % 2
                wait_idx(islot)
                prefetch_idx(b + 1, bb)
                run_block(b, islot, False)
            return carry

        lax.fori_loop(0, npair, pair_body, 0)

        wait_idx(nblk % 2)
        drain_out(0)
        drain_out(1)

    return gather_kernel


def kernel(input, table):
    n = input.size
    idx2d = input.reshape(n // _SUB, _SUB)
    tab4 = table.reshape(table.shape[0] // _GPR, 128)
    out = _build(n)(idx2d, tab4)
    return out.reshape(input.shape + (table.shape[1],))


# speedup vs baseline: 3.3247x; 3.3247x over previous
import functools

import jax
import jax.numpy as jnp
from jax import lax
from jax.experimental import pallas as pl
from jax.experimental.pallas import tpu as pltpu
from jax.experimental.pallas import tpu_sc as plsc

_NC = 2
_NS = 16
_EMB = 32
_L = 200
_B = 4096
_V = 1000000
_KC = 12


@functools.lru_cache(maxsize=None)
def _build():
    mesh = plsc.VectorSubcoreMesh(core_axis_name="c", subcore_axis_name="s",
                                  num_cores=_NC, num_subcores=_NS)
    dpc = _EMB // _NC

    @functools.partial(
        pl.kernel,
        mesh=mesh,
        out_type=(
            jax.ShapeDtypeStruct((_L * _EMB, _B), jnp.float32),
            jax.ShapeDtypeStruct((8, _B), jnp.float32),
        ),
        scratch_types=[
            pltpu.VMEM_SHARED((_V,), jnp.float32),
            pltpu.VMEM((_KC * _B,), jnp.int32),
            pltpu.VMEM((3 * _B,), jnp.float32),
            pltpu.VMEM((_B,), jnp.int32),
            pltpu.SemaphoreType.DMA,
            pltpu.SemaphoreType.DMA,
        ],
    )
    def gather_kernel(idx_hbm, tab_hbm, out_hbm, dump_hbm,
                      shr, idxc, gbuf, idx12, gsem, osem):
        cid = lax.axis_index("c")
        sid = lax.axis_index("s")
        d0 = cid * dpc

        def irow(k):
            return idxc.at[pl.ds(pl.multiple_of(k * _B, _B), _B)]

        def slot(j):
            return gbuf.at[pl.ds(pl.multiple_of(j * _B, _B), _B)]

        for k in range(_KC):
            pltpu.sync_copy(idx_hbm.at[sid + _NS * k], irow(k))

        pltpu.async_copy(slot(0), dump_hbm.at[0], osem)
        pltpu.async_copy(slot(1), dump_hbm.at[1], osem)

        def drain_osem():
            pltpu.make_async_copy(slot(0), dump_hbm.at[0], osem).wait()

        def wait_gsem():
            pltpu.make_async_copy(tab_hbm.at[0, pl.ds(0, _B)], slot(0),
                                  gsem).wait()

        for dd in range(dpc):
            d = d0 + dd

            plsc.subcore_barrier()

            @pl.when(sid == 0)
            def _(d=d):
                pltpu.sync_copy(tab_hbm.at[d], shr)
            plsc.subcore_barrier()

            drain_osem()
            pltpu.async_copy(shr.at[irow(0)], slot(0), gsem)

            def body(k, carry, d=d):
                drain_osem()
                pltpu.async_copy(shr.at[irow(k)], slot(k % 3), gsem)
                wait_gsem()
                row = (sid + _NS * (k - 1)) * _EMB + d
                pltpu.async_copy(slot((k + 2) % 3), out_hbm.at[row], osem)
                return carry

            lax.fori_loop(1, _KC, body, 0)

            wait_gsem()
            row = (sid + _NS * (_KC - 1)) * _EMB + d
            pltpu.async_copy(slot((_KC - 1) % 3), out_hbm.at[row], osem)

            l_last = sid + _NS * _KC

            @pl.when(l_last < _L)
            def _(d=d, l_last=l_last):
                pltpu.sync_copy(idx_hbm.at[l_last], idx12)
                pltpu.async_copy(shr.at[idx12], slot(0), gsem).wait()
                pltpu.sync_copy(slot(0), out_hbm.at[l_last * _EMB + d])

        drain_osem()
        drain_osem()

    return gather_kernel


def kernel(input, table):
    idxP = input.T
    tP = table.T
    out2d, _ = _build()(idxP, tP)
    out3 = out2d.reshape(_L, _EMB, _B)
    return jnp.transpose(out3, (2, 0, 1))
